# baseline (speedup 1.0000x reference)
import functools

import jax
import jax.numpy as jnp
from jax import lax
from jax.experimental import pallas as pl
from jax.experimental.pallas import tpu as pltpu
from jax.experimental.pallas import tpu_sc as plsc

BATCH = 1024
SEQ = 200
EMBED = 100
DIR_VOCAB = 1500

_NC = 2
_NS = 16
_NW = _NC * _NS
_BPW = BATCH // _NW
_PAIRS = _BPW // 2

_EPAD = 128
_COLS = (0, 16, 32, 48, 64, 80, 84)


def _make_body(c):
    def _sc_body(x_hbm, tab_hbm, out_hbm,
                 ib0, ib1, bufa, bufb, cmpa, cmpb, isem, gsem, osem):
        wid = lax.axis_index("s") * _NC + lax.axis_index("c")
        base = wid * _BPW
        ibufs = (ib0, ib1)
        bufs = (bufa, bufb)
        cmps = (cmpa, cmpb)

        def idx_src(b):
            return x_hbm.at[b, pl.ds(2 * c, 2)]

        def issue_gathers(ibuf, s):
            for jj in range(2):
                pltpu.async_copy(tab_hbm.at[ibuf.at[jj]],
                                 bufs[s].at[pl.ds(jj * 100, 100)], gsem.at[s])

        def gwait(s):
            for jj in range(2):
                pltpu.make_async_copy(
                    tab_hbm.at[ib0.at[jj]],
                    bufs[s].at[pl.ds(jj * 100, 100)], gsem.at[s]).wait()

        def iwait(p, b):
            pltpu.make_async_copy(idx_src(b), ibufs[p], isem.at[p]).wait()

        def owait(s, b):
            pltpu.make_async_copy(cmps[s], out_hbm.at[b], osem.at[s]).wait()

        def compact(s):
            def rowcopy(r, cc):
                for col in _COLS:
                    cmps[s][r, pl.ds(col, 16)] = bufs[s][r, pl.ds(col, 16)]
                return cc
            lax.fori_loop(0, SEQ, rowcopy, 0)

        pltpu.async_copy(idx_src(base), ib0, isem.at[0])
        pltpu.async_copy(idx_src(base + 1), ib1, isem.at[1])
        iwait(0, base)
        issue_gathers(ib0, 0)

        def body(k, carry):
            for j in range(2):
                s = j
                b = base + 2 * k + j
                gwait(s)
                @pl.when(k < _PAIRS - 1)
                def _():
                    pltpu.async_copy(idx_src(b + 2), ibufs[s], isem.at[s])
                if j == 0:
                    iwait(1 - s, b + 1)
                    issue_gathers(ibufs[1 - s], 1 - s)
                else:
                    @pl.when(k < _PAIRS - 1)
                    def _():
                        iwait(1 - s, b + 1)
                        issue_gathers(ibufs[1 - s], 1 - s)
                if j < 2:
                    @pl.when(k > 0)
                    def _():
                        owait(s, b)
                compact(s)
                pltpu.async_copy(cmps[s], out_hbm.at[b], osem.at[s])
            return carry

        lax.fori_loop(0, _PAIRS, body, 0)
        owait(0, base)
        owait(1, base)

    return _sc_body


_mesh = plsc.VectorSubcoreMesh(core_axis_name="c", subcore_axis_name="s")

_chan_calls = [
    functools.partial(
        pl.kernel,
        out_type=jax.ShapeDtypeStruct((BATCH, SEQ, EMBED), jnp.float32),
        mesh=_mesh,
        scratch_types=[
            pltpu.VMEM((2, 100), jnp.int32),
            pltpu.VMEM((2, 100), jnp.int32),
            pltpu.VMEM((SEQ, _EPAD), jnp.float32),
            pltpu.VMEM((SEQ, _EPAD), jnp.float32),
            pltpu.VMEM((SEQ, EMBED), jnp.float32),
            pltpu.VMEM((SEQ, EMBED), jnp.float32),
            pltpu.SemaphoreType.DMA((2,)),
            pltpu.SemaphoreType.DMA((2,)),
            pltpu.SemaphoreType.DMA((2,)),
        ],
    )(_make_body(c))
    for c in range(3)
]


def kernel(x, pkt_len_table, iat_table):
    dir_table = jnp.broadcast_to(
        jnp.arange(DIR_VOCAB, dtype=jnp.float32)[:, None], (DIR_VOCAB, _EPAD))
    pad = ((0, 0), (0, _EPAD - EMBED))
    pkt_p = jnp.pad(pkt_len_table, pad)
    iat_p = jnp.pad(iat_table, pad)
    x6 = x.astype(jnp.int32).reshape(BATCH, 6, 100)
    outs = [
        _chan_calls[0](x6, pkt_p),
        _chan_calls[1](x6, dir_table),
        _chan_calls[2](x6, iat_p),
    ]
    return jnp.stack(outs, axis=1)


# speedup vs baseline: 1.3425x; 1.3425x over previous
import functools

import jax
import jax.numpy as jnp
from jax import lax
from jax.experimental import pallas as pl
from jax.experimental.pallas import tpu as pltpu
from jax.experimental.pallas import tpu_sc as plsc

BATCH = 1024
SEQ = 200
EMBED = 100
DIR_VOCAB = 1500

_NC = 2
_NS = 16
_NW = _NC * _NS
_BPW = BATCH // _NW
_PAIRS = _BPW // 2

_EPAD = 128
_COLS = (0, 16, 32, 48, 64, 80, 84)


def _sc_body(x_hbm, pkt_hbm, dir_hbm, iat_hbm, out_hbm,
             ib0, ib1, bufa, bufb, cmpa, cmpb, isem, gsem, osem):
    wid = lax.axis_index("s") * _NC + lax.axis_index("c")
    base = wid * _BPW
    ibufs = (ib0, ib1)
    bufs = (bufa, bufb)
    cmps = (cmpa, cmpb)
    tabs = (pkt_hbm, dir_hbm, iat_hbm)

    def issue_gathers(j, ibuf, s):
        c = j % 3
        for jj in range(2):
            pltpu.async_copy(tabs[c].at[ibuf.at[2 * c + jj]],
                             bufs[s].at[pl.ds(jj * 100, 100)], gsem.at[s])

    def gwait(s):
        for jj in range(2):
            pltpu.make_async_copy(
                pkt_hbm.at[ib0.at[jj]],
                bufs[s].at[pl.ds(jj * 100, 100)], gsem.at[s]).wait()

    def iwait(p, b):
        pltpu.make_async_copy(x_hbm.at[b], ibufs[p], isem.at[p]).wait()

    def owait(s, b):
        pltpu.make_async_copy(cmps[s], out_hbm.at[b, 0], osem.at[s]).wait()

    def compact(s):
        def rowcopy(r4, cc):
            for u in range(4):
                r = r4 * 4 + u
                for col in _COLS:
                    cmps[s][r, pl.ds(col, 16)] = bufs[s][r, pl.ds(col, 16)]
            return cc
        lax.fori_loop(0, SEQ // 4, rowcopy, 0)

    pltpu.async_copy(x_hbm.at[base], ib0, isem.at[0])
    pltpu.async_copy(x_hbm.at[base + 1], ib1, isem.at[1])
    iwait(0, base)
    issue_gathers(0, ib0, 0)

    def body(k, carry):
        b0 = base + 2 * k
        b1 = b0 + 1
        for j in range(6):
            s = j % 2
            bj = b0 if j < 3 else b1
            gwait(s)
            if j == 2:
                @pl.when(k < _PAIRS - 1)
                def _():
                    pltpu.async_copy(x_hbm.at[b0 + 2], ib0, isem.at[0])
                iwait(1, b1)
            if j == 5:
                @pl.when(k < _PAIRS - 1)
                def _():
                    pltpu.async_copy(x_hbm.at[b1 + 2], ib1, isem.at[1])
            if j < 5:
                issue_gathers(j + 1, ib0 if j + 1 < 3 else ib1, 1 - s)
            else:
                @pl.when(k < _PAIRS - 1)
                def _():
                    iwait(0, b0 + 2)
                    issue_gathers(0, ib0, 1 - s)
            if j < 2:
                @pl.when(k > 0)
                def _():
                    owait(s, b0)
            else:
                owait(s, b0)
            compact(s)
            pltpu.async_copy(cmps[s], out_hbm.at[bj, j % 3], osem.at[s])
        return carry

    lax.fori_loop(0, _PAIRS, body, 0)
    owait(0, base)
    owait(1, base)


_mesh = plsc.VectorSubcoreMesh(core_axis_name="c", subcore_axis_name="s")

_gather_all = functools.partial(
    pl.kernel,
    out_type=jax.ShapeDtypeStruct((BATCH, 3, SEQ, EMBED), jnp.float32),
    mesh=_mesh,
    scratch_types=[
        pltpu.VMEM((6, 100), jnp.int32),
        pltpu.VMEM((6, 100), jnp.int32),
        pltpu.VMEM((SEQ, _EPAD), jnp.float32),
        pltpu.VMEM((SEQ, _EPAD), jnp.float32),
        pltpu.VMEM((SEQ, EMBED), jnp.float32),
        pltpu.VMEM((SEQ, EMBED), jnp.float32),
        pltpu.SemaphoreType.DMA((2,)),
        pltpu.SemaphoreType.DMA((2,)),
        pltpu.SemaphoreType.DMA((2,)),
    ],
)(_sc_body)


def kernel(x, pkt_len_table, iat_table):
    dir_table = jnp.broadcast_to(
        jnp.arange(DIR_VOCAB, dtype=jnp.float32)[:, None], (DIR_VOCAB, _EPAD))
    pad = ((0, 0), (0, _EPAD - EMBED))
    pkt_p = jnp.pad(pkt_len_table, pad)
    iat_p = jnp.pad(iat_table, pad)
    x6 = x.astype(jnp.int32).reshape(BATCH, 6, 100)
    return _gather_all(x6, pkt_p, dir_table, iat_p)


# speedup vs baseline: 1.5062x; 1.1219x over previous
import functools

import jax
import jax.numpy as jnp
from jax import lax
from jax.experimental import pallas as pl
from jax.experimental.pallas import tpu as pltpu
from jax.experimental.pallas import tpu_sc as plsc

BATCH = 1024
SEQ = 200
EMBED = 100
DIR_VOCAB = 1500

_NC = 2
_NS = 16
_NW = _NC * _NS
_BPW = BATCH // _NW
_PAIRS = _BPW // 2

_EPAD = 128
_COLS = (0, 16, 32, 48, 64, 80, 84)


def _sc_body(x_hbm, comb_hbm, iat_hbm, out_hbm,
             ib0, ib1, bufa, bufb, cmpa, cmpb, spm, isem, gsem, osem):
    wid = lax.axis_index("s") * _NC + lax.axis_index("c")
    base = wid * _BPW
    ibufs = (ib0, ib1)
    bufs = (bufa, bufb)
    cmps = (cmpa, cmpb)

    @pl.when(lax.axis_index("s") == 0)
    def _():
        pltpu.sync_copy(comb_hbm, spm)
    plsc.subcore_barrier()

    def issue_gathers(j, ibuf, s):
        c = j % 3
        tab = iat_hbm if c == 2 else spm
        for jj in range(2):
            pltpu.async_copy(tab.at[ibuf.at[2 * c + jj]],
                             bufs[s].at[pl.ds(jj * 100, 100)], gsem.at[s])

    def gwait(s):
        for jj in range(2):
            pltpu.make_async_copy(
                spm.at[ib0.at[jj]],
                bufs[s].at[pl.ds(jj * 100, 100)], gsem.at[s]).wait()

    def iwait(p, b):
        pltpu.make_async_copy(x_hbm.at[b], ibufs[p], isem.at[p]).wait()

    def owait(s, b):
        pltpu.make_async_copy(cmps[s], out_hbm.at[b, 0], osem.at[s]).wait()

    def compact(s):
        def rowcopy(r4, cc):
            for u in range(4):
                r = r4 * 4 + u
                for col in _COLS:
                    cmps[s][r, pl.ds(col, 16)] = bufs[s][r, pl.ds(col, 16)]
            return cc
        lax.fori_loop(0, SEQ // 4, rowcopy, 0)

    pltpu.async_copy(x_hbm.at[base], ib0, isem.at[0])
    pltpu.async_copy(x_hbm.at[base + 1], ib1, isem.at[1])
    iwait(0, base)
    issue_gathers(0, ib0, 0)

    def body(k, carry):
        b0 = base + 2 * k
        b1 = b0 + 1
        for j in range(6):
            s = j % 2
            bj = b0 if j < 3 else b1
            gwait(s)
            if j == 2:
                @pl.when(k < _PAIRS - 1)
                def _():
                    pltpu.async_copy(x_hbm.at[b0 + 2], ib0, isem.at[0])
                iwait(1, b1)
            if j == 5:
                @pl.when(k < _PAIRS - 1)
                def _():
                    pltpu.async_copy(x_hbm.at[b1 + 2], ib1, isem.at[1])
            if j < 5:
                issue_gathers(j + 1, ib0 if j + 1 < 3 else ib1, 1 - s)
            else:
                @pl.when(k < _PAIRS - 1)
                def _():
                    iwait(0, b0 + 2)
                    issue_gathers(0, ib0, 1 - s)
            if j < 2:
                @pl.when(k > 0)
                def _():
                    owait(s, b0)
            else:
                owait(s, b0)
            compact(s)
            pltpu.async_copy(cmps[s], out_hbm.at[bj, j % 3], osem.at[s])
        return carry

    lax.fori_loop(0, _PAIRS, body, 0)
    owait(0, base)
    owait(1, base)


_mesh = plsc.VectorSubcoreMesh(core_axis_name="c", subcore_axis_name="s")

_gather_all = functools.partial(
    pl.kernel,
    out_type=jax.ShapeDtypeStruct((BATCH, 3, SEQ, EMBED), jnp.float32),
    mesh=_mesh,
    scratch_types=[
        pltpu.VMEM((6, 100), jnp.int32),
        pltpu.VMEM((6, 100), jnp.int32),
        pltpu.VMEM((SEQ, _EPAD), jnp.float32),
        pltpu.VMEM((SEQ, _EPAD), jnp.float32),
        pltpu.VMEM((SEQ, EMBED), jnp.float32),
        pltpu.VMEM((SEQ, EMBED), jnp.float32),
        pltpu.VMEM_SHARED((2 * DIR_VOCAB, _EPAD), jnp.float32),
        pltpu.SemaphoreType.DMA((2,)),
        pltpu.SemaphoreType.DMA((2,)),
        pltpu.SemaphoreType.DMA((2,)),
    ],
)(_sc_body)


def kernel(x, pkt_len_table, iat_table):
    dir_table = jnp.broadcast_to(
        jnp.arange(DIR_VOCAB, dtype=jnp.float32)[:, None], (DIR_VOCAB, _EPAD))
    pad = ((0, 0), (0, _EPAD - EMBED))
    pkt_p = jnp.pad(pkt_len_table, pad)
    iat_p = jnp.pad(iat_table, pad)
    comb = jnp.concatenate([pkt_p, dir_table], axis=0)
    offs = jnp.array([0, 0, DIR_VOCAB, DIR_VOCAB, 0, 0],
                     jnp.int32)[None, :, None]
    x6 = x.astype(jnp.int32).reshape(BATCH, 6, 100) + offs
    return _gather_all(x6, comb, iat_p)


# speedup vs baseline: 1.7008x; 1.1292x over previous
import functools

import jax
import jax.numpy as jnp
from jax import lax
from jax.experimental import pallas as pl
from jax.experimental.pallas import tpu as pltpu
from jax.experimental.pallas import tpu_sc as plsc

BATCH = 1024
SEQ = 200
EMBED = 100
DIR_VOCAB = 1500

_NC = 2
_NS = 16
_NW = _NC * _NS
_BPW = BATCH // _NW
_PAIRS = _BPW // 2

_EPAD = 128
_COLS = (0, 16, 32, 48, 64, 80, 84)


def _sc_body(x_hbm, comb_hbm, iat_hbm, out_hbm,
             ib0, ib1, bufa, bufb, cmpa, cmpb, spm, isem, gsem, osem):
    wid = lax.axis_index("s") * _NC + lax.axis_index("c")
    base = wid * _BPW
    ibufs = (ib0, ib1)
    bufs = (bufa, bufb)
    cmps = (cmpa, cmpb)

    @pl.when(lax.axis_index("s") == 0)
    def _():
        pltpu.sync_copy(comb_hbm, spm)
    plsc.subcore_barrier()

    def issue_gathers(j, ibuf, s):
        c = j % 3
        tab = iat_hbm if c == 2 else spm
        for jj in range(2):
            pltpu.async_copy(tab.at[ibuf.at[2 * c + jj]],
                             bufs[s].at[pl.ds(jj * 100, 100)], gsem.at[s])

    def gwait(s):
        for jj in range(2):
            pltpu.make_async_copy(
                spm.at[ib0.at[jj]],
                bufs[s].at[pl.ds(jj * 100, 100)], gsem.at[s]).wait()

    def iwait(p, b):
        pltpu.make_async_copy(x_hbm.at[b], ibufs[p], isem.at[p]).wait()

    def owait(s, b):
        pltpu.make_async_copy(cmps[s], out_hbm.at[b, 0], osem.at[s]).wait()

    def compact(s):
        def rowcopy(r4, cc):
            for u in range(4):
                r = r4 * 4 + u
                for col in _COLS:
                    cmps[s][r, pl.ds(col, 16)] = bufs[s][r, pl.ds(col, 16)]
            return cc
        lax.fori_loop(0, SEQ // 4, rowcopy, 0)

    pltpu.async_copy(x_hbm.at[base], ib0, isem.at[0])
    pltpu.async_copy(x_hbm.at[base + 1], ib1, isem.at[1])
    iwait(0, base)
    issue_gathers(0, ib0, 0)

    def body(k, carry):
        b0 = base + 2 * k
        b1 = b0 + 1
        for j in range(6):
            s = j % 2
            bj = b0 if j < 3 else b1
            gwait(s)
            if j == 2:
                @pl.when(k < _PAIRS - 1)
                def _():
                    pltpu.async_copy(x_hbm.at[b0 + 2], ib0, isem.at[0])
                iwait(1, b1)
            if j == 5:
                @pl.when(k < _PAIRS - 1)
                def _():
                    pltpu.async_copy(x_hbm.at[b1 + 2], ib1, isem.at[1])
            if j < 5:
                issue_gathers(j + 1, ib0 if j + 1 < 3 else ib1, 1 - s)
            else:
                @pl.when(k < _PAIRS - 1)
                def _():
                    iwait(0, b0 + 2)
                    issue_gathers(0, ib0, 1 - s)
            if j < 2:
                @pl.when(k > 0)
                def _():
                    owait(s, b0)
            else:
                owait(s, b0)
            compact(s)
            pltpu.async_copy(cmps[s], out_hbm.at[bj, j % 3], osem.at[s])
        return carry

    lax.fori_loop(0, _PAIRS, body, 0)
    owait(0, base)
    owait(1, base)


_mesh = plsc.VectorSubcoreMesh(core_axis_name="c", subcore_axis_name="s")

_gather_all = functools.partial(
    pl.kernel,
    out_type=jax.ShapeDtypeStruct((BATCH, 3, SEQ, EMBED), jnp.float32),
    mesh=_mesh,
    scratch_types=[
        pltpu.VMEM((6, 100), jnp.int32),
        pltpu.VMEM((6, 100), jnp.int32),
        pltpu.VMEM((SEQ, _EPAD), jnp.float32),
        pltpu.VMEM((SEQ, _EPAD), jnp.float32),
        pltpu.VMEM((SEQ, EMBED), jnp.float32),
        pltpu.VMEM((SEQ, EMBED), jnp.float32),
        pltpu.VMEM_SHARED((2 * DIR_VOCAB, _EPAD), jnp.float32),
        pltpu.SemaphoreType.DMA((2,)),
        pltpu.SemaphoreType.DMA((2,)),
        pltpu.SemaphoreType.DMA((2,)),
    ],
)(_sc_body)


def kernel(x, pkt_len_table, iat_table):
    dir_table = jnp.broadcast_to(
        jnp.arange(DIR_VOCAB, dtype=jnp.float32)[:, None], (DIR_VOCAB, _EPAD))
    pad = ((0, 0), (0, _EPAD - EMBED))
    pkt_p = jnp.pad(pkt_len_table, pad)
    iat_p = jnp.pad(iat_table, pad)
    comb = jnp.concatenate([pkt_p, dir_table], axis=0)
    offs = jnp.array([0, 0, DIR_VOCAB, DIR_VOCAB, 0, 0],
                     jnp.int32)[None, :, None]
    x6 = x.astype(jnp.int32).reshape(BATCH, 6, 100) + offs
    out = _gather_all(x6, comb, iat_p)
    return lax.optimization_barrier(out)
